# baseline (speedup 1.0000x reference)
import functools

import jax
import jax.numpy as jnp
from jax import lax
from jax.experimental import pallas as pl
from jax.experimental.pallas import tpu as pltpu
from jax.experimental.pallas import tpu_sc as plsc

_BN_THR = 0.5
_B, _C, _H, _W = 8, 192, 128, 128
_ROW = _H * _W
_NROWS = _B * _C
_NC, _NS = 2, 16
_NW = _NC * _NS
_RPW = _NROWS // _NW
_LANES = 16


def _exchange_body(x0, x1, bn1, bn2, y1, y2, bn_v, m1_v, m2_v, buf_a, buf_b):
    wid = lax.axis_index("s") * _NC + lax.axis_index("c")

    pltpu.sync_copy(bn1, bn_v)
    for g in range(_C // _LANES):
        w = bn_v[pl.ds(g * _LANES, _LANES)]
        m = jnp.where(jnp.abs(w) >= _BN_THR,
                      jnp.full((_LANES,), 1, jnp.int32),
                      jnp.full((_LANES,), 0, jnp.int32))
        m1_v[pl.ds(g * _LANES, _LANES)] = m
    pltpu.sync_copy(bn2, bn_v)
    for g in range(_C // _LANES):
        w = bn_v[pl.ds(g * _LANES, _LANES)]
        m = jnp.where(jnp.abs(w) >= _BN_THR,
                      jnp.full((_LANES,), 1, jnp.int32),
                      jnp.full((_LANES,), 0, jnp.int32))
        m2_v[pl.ds(g * _LANES, _LANES)] = m

    base = wid * _RPW

    @pl.loop(0, _RPW)
    def _(i):
        r = base + i
        c = lax.rem(r, _C)
        s1 = m1_v[pl.ds(c, _LANES)][0]
        s2 = m2_v[pl.ds(c, _LANES)][0]

        @pl.when(s1 == 1)
        def _():
            pltpu.sync_copy(x0.at[r], buf_a)

        @pl.when(s1 == 0)
        def _():
            pltpu.sync_copy(x1.at[r], buf_a)

        pltpu.sync_copy(buf_a, y1.at[r])

        @pl.when(s2 == 1)
        def _():
            pltpu.sync_copy(x1.at[r], buf_b)

        @pl.when(s2 == 0)
        def _():
            pltpu.sync_copy(x0.at[r], buf_b)

        pltpu.sync_copy(buf_b, y2.at[r])


_exchange = pl.kernel(
    _exchange_body,
    out_type=(
        jax.ShapeDtypeStruct((_NROWS, _ROW), jnp.float32),
        jax.ShapeDtypeStruct((_NROWS, _ROW), jnp.float32),
    ),
    mesh=plsc.VectorSubcoreMesh(
        core_axis_name="c", subcore_axis_name="s",
        num_cores=_NC, num_subcores=_NS),
    scratch_types=[
        pltpu.VMEM((_C,), jnp.float32),
        pltpu.VMEM((_C + _LANES,), jnp.int32),
        pltpu.VMEM((_C + _LANES,), jnp.int32),
        pltpu.VMEM((_ROW,), jnp.float32),
        pltpu.VMEM((_ROW,), jnp.float32),
    ],
)


def kernel(x0, x1, bn1_weight, bn2_weight):
    x0r = x0.reshape(_NROWS, _ROW)
    x1r = x1.reshape(_NROWS, _ROW)
    y1, y2 = _exchange(x0r, x1r, bn1_weight, bn2_weight)
    return (y1.reshape(_B, _C, _H, _W), y2.reshape(_B, _C, _H, _W))


# speedup vs baseline: 1.1322x; 1.1322x over previous
import functools

import jax
import jax.numpy as jnp
from jax import lax
from jax.experimental import pallas as pl
from jax.experimental.pallas import tpu as pltpu
from jax.experimental.pallas import tpu_sc as plsc

_BN_THR = 0.5
_B, _C, _H, _W = 8, 192, 128, 128
_ROW = _H * _W
_NROWS = _B * _C
_NC, _NS = 2, 16
_NW = _NC * _NS
_RPW = _NROWS // _NW
_LANES = 16


def _exchange_body(x0, x1, bn1, bn2, y1, y2,
                   bn_v, m1_v, m2_v,
                   buf_a0, buf_b0, buf_a1, buf_b1,
                   gsem0, gsem1, ssem0, ssem1):
    wid = lax.axis_index("s") * _NC + lax.axis_index("c")

    pltpu.sync_copy(bn1, bn_v)
    for g in range(_C // _LANES):
        w = bn_v[pl.ds(g * _LANES, _LANES)]
        m = jnp.where(jnp.abs(w) >= _BN_THR,
                      jnp.full((_LANES,), 1, jnp.int32),
                      jnp.full((_LANES,), 0, jnp.int32))
        m1_v[pl.ds(g * _LANES, _LANES)] = m
    pltpu.sync_copy(bn2, bn_v)
    for g in range(_C // _LANES):
        w = bn_v[pl.ds(g * _LANES, _LANES)]
        m = jnp.where(jnp.abs(w) >= _BN_THR,
                      jnp.full((_LANES,), 1, jnp.int32),
                      jnp.full((_LANES,), 0, jnp.int32))
        m2_v[pl.ds(g * _LANES, _LANES)] = m

    base = wid * _RPW
    bufs = ((buf_a0, buf_b0), (buf_a1, buf_b1))
    gsems = (gsem0, gsem1)
    ssems = (ssem0, ssem1)

    def issue_gathers(i, slot):
        r = base + i
        c = lax.rem(r, _C)
        s1 = m1_v[pl.ds(c, _LANES)][0]
        s2 = m2_v[pl.ds(c, _LANES)][0]
        buf_a, buf_b = bufs[slot]
        gsem = gsems[slot]

        @pl.when(s1 == 1)
        def _():
            pltpu.async_copy(x0.at[r], buf_a, gsem)

        @pl.when(s1 == 0)
        def _():
            pltpu.async_copy(x1.at[r], buf_a, gsem)

        @pl.when(s2 == 1)
        def _():
            pltpu.async_copy(x1.at[r], buf_b, gsem)

        @pl.when(s2 == 0)
        def _():
            pltpu.async_copy(x0.at[r], buf_b, gsem)

    def wait_gathers(slot):
        buf_a, buf_b = bufs[slot]
        pltpu.make_async_copy(x0.at[0], buf_a, gsems[slot]).wait()
        pltpu.make_async_copy(x0.at[0], buf_b, gsems[slot]).wait()

    def issue_scatters(i, slot):
        r = base + i
        buf_a, buf_b = bufs[slot]
        pltpu.async_copy(buf_a, y1.at[r], ssems[slot])
        pltpu.async_copy(buf_b, y2.at[r], ssems[slot])

    def wait_scatters(slot):
        buf_a, buf_b = bufs[slot]
        pltpu.make_async_copy(x0.at[0], buf_a, ssems[slot]).wait()
        pltpu.make_async_copy(x0.at[0], buf_b, ssems[slot]).wait()

    issue_gathers(0, 0)

    @pl.loop(0, _RPW // 2)
    def _(j):
        i0 = 2 * j

        wait_gathers(0)
        issue_scatters(i0, 0)

        @pl.when(j >= 1)
        def _():
            wait_scatters(1)

        issue_gathers(i0 + 1, 1)

        wait_gathers(1)
        issue_scatters(i0 + 1, 1)
        wait_scatters(0)

        @pl.when(j < _RPW // 2 - 1)
        def _():
            issue_gathers(i0 + 2, 0)

    wait_scatters(1)


_exchange = pl.kernel(
    _exchange_body,
    out_type=(
        jax.ShapeDtypeStruct((_NROWS, _ROW), jnp.float32),
        jax.ShapeDtypeStruct((_NROWS, _ROW), jnp.float32),
    ),
    mesh=plsc.VectorSubcoreMesh(
        core_axis_name="c", subcore_axis_name="s",
        num_cores=_NC, num_subcores=_NS),
    scratch_types=[
        pltpu.VMEM((_C,), jnp.float32),
        pltpu.VMEM((_C + _LANES,), jnp.int32),
        pltpu.VMEM((_C + _LANES,), jnp.int32),
        pltpu.VMEM((_ROW,), jnp.float32),
        pltpu.VMEM((_ROW,), jnp.float32),
        pltpu.VMEM((_ROW,), jnp.float32),
        pltpu.VMEM((_ROW,), jnp.float32),
        pltpu.SemaphoreType.DMA,
        pltpu.SemaphoreType.DMA,
        pltpu.SemaphoreType.DMA,
        pltpu.SemaphoreType.DMA,
    ],
)


def kernel(x0, x1, bn1_weight, bn2_weight):
    x0r = x0.reshape(_NROWS, _ROW)
    x1r = x1.reshape(_NROWS, _ROW)
    y1, y2 = _exchange(x0r, x1r, bn1_weight, bn2_weight)
    return (y1.reshape(_B, _C, _H, _W), y2.reshape(_B, _C, _H, _W))
